# baseline (speedup 1.0000x reference)
import dataclasses
import functools

import jax
import jax.numpy as jnp
from jax import lax
from jax.experimental import pallas as pl
from jax.experimental.pallas import tpu as pltpu
from jax.experimental.pallas import tpu_sc as plsc

N = 10000
E = 320000
D = 128
L = 5
G = 64
S = 512

NC = 2
NS = 16
NW = NC * NS

EDGE_CHUNK = 96
CHUNKS_PER_TILE = 105
E_PAD = NW * CHUNKS_PER_TILE * EDGE_CHUNK
ROW_CHUNK = 80
N_ROW_CHUNKS = N // ROW_CHUNK
N_ACC = N + 8

POOL_CHUNK = 80
N_POOL_CHUNKS = N // POOL_CHUNK
POOL_ROWS_PER_SUBCORE = S // NS

_mesh = plsc.VectorSubcoreMesh(core_axis_name="c", subcore_axis_name="s")

_sc_params = pltpu.CompilerParams()
if "needs_layout_passes" in pltpu.CompilerParams.__dataclass_fields__:
    _sc_params = dataclasses.replace(_sc_params, needs_layout_passes=False)


def _zero_vmem_2d(ref, rows, cols):
    z = jnp.zeros((16,), jnp.float32)

    @pl.loop(0, rows)
    def _(r):
        @pl.loop(0, cols // 16)
        def _(c):
            ref[r, pl.ds(c * 16, 16)] = z


@functools.partial(
    pl.kernel,
    out_type=jax.ShapeDtypeStruct((NC, N, D), jnp.float32),
    mesh=_mesh,
    scratch_types=[
        pltpu.VMEM((CHUNKS_PER_TILE * EDGE_CHUNK,), jnp.int32),
        pltpu.VMEM((EDGE_CHUNK,), jnp.int32),
        pltpu.VMEM((EDGE_CHUNK,), jnp.int32),
        pltpu.VMEM((EDGE_CHUNK,), jnp.int32),
        pltpu.VMEM((EDGE_CHUNK, D), jnp.float32),
        pltpu.VMEM((EDGE_CHUNK, D), jnp.float32),
        pltpu.VMEM((EDGE_CHUNK, D), jnp.float32),
        pltpu.VMEM_SHARED((N_ACC, D), jnp.float32),
    ] + [pltpu.SemaphoreType.DMA] * 9,
)
def _sc_segment_sum(h_hbm, src_hbm, dst_hbm, out_hbm,
                    src_v, d0, d1, d2, r0, r1, r2, acc_sh,
                    sg0, sg1, sg2, ss0, ss1, ss2, sd0, sd1, sd2):
    cid = lax.axis_index("c")
    sid = lax.axis_index("s")
    wid = sid * NC + cid
    tbase = wid * CHUNKS_PER_TILE

    pltpu.sync_copy(
        src_hbm.at[pl.ds(wid * CHUNKS_PER_TILE * EDGE_CHUNK,
                         CHUNKS_PER_TILE * EDGE_CHUNK)], src_v)

    _zero_vmem_2d(r0, ROW_CHUNK, D)

    @pl.loop(sid, N_ROW_CHUNKS, step=NS)
    def _(j):
        pltpu.sync_copy(r0.at[pl.ds(0, ROW_CHUNK)],
                        acc_sh.at[pl.ds(j * ROW_CHUNK, ROW_CHUNK)])

    plsc.subcore_barrier()

    def _sidx(j):
        return src_v.at[pl.ds(j * EDGE_CHUNK, EDGE_CHUNK)]

    def _dslice(j):
        return dst_hbm.at[pl.ds((tbase + j) * EDGE_CHUNK, EDGE_CHUNK)]

    slots = ((r0, d0, sg0, ss0, sd0),
             (r1, d1, sg1, ss1, sd1),
             (r2, d2, sg2, ss2, sd2))

    def _start_load(j, r, d, sg, sd):
        pltpu.async_copy(_dslice(j), d, sd)
        pltpu.async_copy(h_hbm.at[_sidx(j)], r, sg)

    def _finish_and_scatter(j, r, d, sg, ss, sd):
        pltpu.make_async_copy(h_hbm.at[_sidx(j)], r, sg).wait()
        pltpu.make_async_copy(_dslice(j), d, sd).wait()
        pltpu.async_copy(r, acc_sh.at[d], ss, add=True)

    for o, (r, d, sg, ss, sd) in enumerate(slots):
        _start_load(o, r, d, sg, sd)

    @pl.loop(0, CHUNKS_PER_TILE // 3 - 1)
    def _(t):
        j0 = 3 * t
        for o, (r, d, sg, ss, sd) in enumerate(slots):
            _finish_and_scatter(j0 + o, r, d, sg, ss, sd)
        for o, (r, d, sg, ss, sd) in enumerate(slots):
            pltpu.make_async_copy(r, acc_sh.at[d], ss).wait()
            _start_load(j0 + 3 + o, r, d, sg, sd)

    jlast = CHUNKS_PER_TILE - 3
    for o, (r, d, sg, ss, sd) in enumerate(slots):
        _finish_and_scatter(jlast + o, r, d, sg, ss, sd)
    for o, (r, d, sg, ss, sd) in enumerate(slots):
        pltpu.make_async_copy(r, acc_sh.at[d], ss).wait()

    plsc.subcore_barrier()

    @pl.loop(sid, N_ROW_CHUNKS, step=NS)
    def _(j):
        pltpu.sync_copy(acc_sh.at[pl.ds(j * ROW_CHUNK, ROW_CHUNK)],
                        out_hbm.at[cid, pl.ds(j * ROW_CHUNK, ROW_CHUNK)])


def _tc_layer_body(h_ref, p_ref, wrel_ref, wroot_ref, brel_ref,
                   gamma_ref, beta_ref, o_ref):
    agg = p_ref[0] + p_ref[1]
    out = (jnp.dot(agg, wrel_ref[...], preferred_element_type=jnp.float32)
           + jnp.dot(h_ref[...], wroot_ref[...],
                     preferred_element_type=jnp.float32)
           + brel_ref[...])
    mu = jnp.mean(out, axis=0, keepdims=True)
    var = jnp.mean((out - mu) ** 2, axis=0, keepdims=True)
    normed = (out - mu) * lax.rsqrt(var + 1e-5) * gamma_ref[...] + beta_ref[...]
    o_ref[...] = jnp.maximum(normed, 0.0)


_tc_layer = pl.pallas_call(
    _tc_layer_body,
    out_shape=jax.ShapeDtypeStruct((N, D), jnp.float32),
)


@functools.partial(
    pl.kernel,
    out_type=[jax.ShapeDtypeStruct((NC, S, D), jnp.float32),
              jax.ShapeDtypeStruct((NC, S, D), jnp.float32)],
    mesh=_mesh,
    scratch_types=[
        pltpu.VMEM((G,), jnp.int32),
        pltpu.VMEM((G,), jnp.int32),
        pltpu.VMEM((POOL_CHUNK,), jnp.int32),
        pltpu.VMEM((POOL_CHUNK,), jnp.int32),
        pltpu.VMEM((POOL_CHUNK,), jnp.int32),
        pltpu.VMEM((POOL_CHUNK, D), jnp.float32),
        pltpu.VMEM((POOL_CHUNK, D), jnp.float32),
        pltpu.VMEM_SHARED((S, D), jnp.float32),
        pltpu.VMEM_SHARED((S, D), jnp.float32),
    ],
    compiler_params=_sc_params,
)
def _sc_pool(h_hbm, batch_hbm, sb_hbm, ns_hbm, sum_hbm, cnt_hbm,
             ns_v, offs_v, bt_v, sb_v, id_v, rows_v, ones_v,
             acc_sh, cnt_sh):
    cid = lax.axis_index("c")
    sid = lax.axis_index("s")
    wid = sid * NC + cid

    pltpu.sync_copy(ns_hbm, ns_v)
    carry = jnp.int32(0)
    for k in range(G // 16):
        v = ns_v[pl.ds(k * 16, 16)]
        incl = plsc.cumsum(v)
        offs_v[pl.ds(k * 16, 16)] = incl - v + carry
        carry = carry + jnp.sum(v)

    _zero_vmem_2d(rows_v, POOL_CHUNK, D)
    one = jnp.ones((16,), jnp.float32)

    @pl.loop(0, POOL_CHUNK)
    def _(r):
        @pl.loop(0, D // 16)
        def _(c):
            ones_v[r, pl.ds(c * 16, 16)] = one

    pbase = sid * POOL_ROWS_PER_SUBCORE
    pltpu.sync_copy(rows_v.at[pl.ds(0, POOL_ROWS_PER_SUBCORE)],
                    acc_sh.at[pl.ds(pbase, POOL_ROWS_PER_SUBCORE)])
    pltpu.sync_copy(rows_v.at[pl.ds(0, POOL_ROWS_PER_SUBCORE)],
                    cnt_sh.at[pl.ds(pbase, POOL_ROWS_PER_SUBCORE)])
    plsc.subcore_barrier()

    @pl.loop(wid, N_POOL_CHUNKS, step=NW)
    def _(i):
        nbase = i * POOL_CHUNK
        pltpu.sync_copy(batch_hbm.at[pl.ds(nbase, POOL_CHUNK)], bt_v)
        pltpu.sync_copy(sb_hbm.at[pl.ds(nbase, POOL_CHUNK)], sb_v)
        for k in range(POOL_CHUNK // 16):
            idx16 = bt_v[pl.ds(k * 16, 16)]
            off16 = plsc.load_gather(offs_v, [idx16])
            id_v[pl.ds(k * 16, 16)] = sb_v[pl.ds(k * 16, 16)] + off16
        pltpu.sync_copy(h_hbm.at[pl.ds(nbase, POOL_CHUNK)], rows_v)
        pltpu.sync_copy(rows_v, acc_sh.at[id_v], add=True)
        pltpu.sync_copy(ones_v, cnt_sh.at[id_v], add=True)

    plsc.subcore_barrier()
    pltpu.sync_copy(acc_sh.at[pl.ds(pbase, POOL_ROWS_PER_SUBCORE)],
                    sum_hbm.at[cid, pl.ds(pbase, POOL_ROWS_PER_SUBCORE)])
    pltpu.sync_copy(cnt_sh.at[pl.ds(pbase, POOL_ROWS_PER_SUBCORE)],
                    cnt_hbm.at[cid, pl.ds(pbase, POOL_ROWS_PER_SUBCORE)])


def _tc_finalize_body(s_ref, c_ref, o_ref):
    s = s_ref[0] + s_ref[1]
    c = c_ref[0] + c_ref[1]
    o_ref[...] = s / jnp.maximum(c[:, 0:1], 1.0)


_tc_finalize = pl.pallas_call(
    _tc_finalize_body,
    out_shape=jax.ShapeDtypeStruct((S, D), jnp.float32),
)


def kernel(x, edge_index, edge_attr, batch, num_subgraphs, subgraph_batch,
           Wroot, Wrel, brel, gamma, beta):
    pad = E_PAD - E
    src = jnp.concatenate([edge_index[0], jnp.zeros((pad,), jnp.int32)])
    dst = jnp.concatenate([edge_index[1], jnp.full((pad,), N, jnp.int32)])
    h = x
    for l in range(L):
        partials = _sc_segment_sum(h, src, dst)
        h = _tc_layer(h, partials, Wrel[l], Wroot[l],
                      brel[l].reshape(1, D), gamma[l].reshape(1, D),
                      beta[l].reshape(1, D))
    sums, cnts = _sc_pool(h, batch, subgraph_batch, num_subgraphs)
    return _tc_finalize(sums, cnts)


# speedup vs baseline: 1.0051x; 1.0051x over previous
import dataclasses
import functools

import jax
import jax.numpy as jnp
from jax import lax
from jax.experimental import pallas as pl
from jax.experimental.pallas import tpu as pltpu
from jax.experimental.pallas import tpu_sc as plsc

N = 10000
E = 320000
D = 128
L = 5
G = 64
S = 512

NC = 2
NS = 16
NW = NC * NS

EDGE_CHUNK = 72
CHUNKS_PER_TILE = 140
E_PAD = NW * CHUNKS_PER_TILE * EDGE_CHUNK
ROW_CHUNK = 40
N_ROW_CHUNKS = N // ROW_CHUNK
OUT_CHUNK = 200
N_OUT_CHUNKS = N // OUT_CHUNK
N_ACC = N + 8

POOL_CHUNK = 80
N_POOL_CHUNKS = N // POOL_CHUNK
POOL_ROWS_PER_SUBCORE = S // NS

_mesh = plsc.VectorSubcoreMesh(core_axis_name="c", subcore_axis_name="s")

_sc_params = pltpu.CompilerParams()
if "needs_layout_passes" in pltpu.CompilerParams.__dataclass_fields__:
    _sc_params = dataclasses.replace(_sc_params, needs_layout_passes=False)


def _zero_vmem_2d(ref, rows, cols):
    z = jnp.zeros((16,), jnp.float32)

    @pl.loop(0, rows)
    def _(r):
        @pl.loop(0, cols // 16)
        def _(c):
            ref[r, pl.ds(c * 16, 16)] = z


@functools.partial(
    pl.kernel,
    out_type=jax.ShapeDtypeStruct((NC, N, D), jnp.float32),
    mesh=_mesh,
    scratch_types=[
        pltpu.VMEM((CHUNKS_PER_TILE * EDGE_CHUNK,), jnp.int32),
        pltpu.VMEM((EDGE_CHUNK,), jnp.int32),
        pltpu.VMEM((EDGE_CHUNK,), jnp.int32),
        pltpu.VMEM((EDGE_CHUNK,), jnp.int32),
        pltpu.VMEM((EDGE_CHUNK,), jnp.int32),
        pltpu.VMEM((EDGE_CHUNK, D), jnp.float32),
        pltpu.VMEM((EDGE_CHUNK, D), jnp.float32),
        pltpu.VMEM((EDGE_CHUNK, D), jnp.float32),
        pltpu.VMEM((EDGE_CHUNK, D), jnp.float32),
        pltpu.VMEM_SHARED((N_ACC, D), jnp.float32),
    ] + [pltpu.SemaphoreType.DMA] * 12,
)
def _sc_segment_sum(h_hbm, src_hbm, dst_hbm, out_hbm,
                    src_v, d0, d1, d2, d3, r0, r1, r2, r3, acc_sh,
                    sg0, sg1, sg2, sg3, ss0, ss1, ss2, ss3,
                    sd0, sd1, sd2, sd3):
    cid = lax.axis_index("c")
    sid = lax.axis_index("s")
    wid = sid * NC + cid
    tbase = wid * CHUNKS_PER_TILE

    def _sidx(j):
        return src_v.at[pl.ds(j * EDGE_CHUNK, EDGE_CHUNK)]

    def _dslice(j):
        return dst_hbm.at[pl.ds((tbase + j) * EDGE_CHUNK, EDGE_CHUNK)]

    slots = ((r0, d0, sg0, ss0, sd0),
             (r1, d1, sg1, ss1, sd1),
             (r2, d2, sg2, ss2, sd2),
             (r3, d3, sg3, ss3, sd3))

    def _start_load(j, o):
        r, d, sg, ss, sd = slots[o]
        pltpu.async_copy(_dslice(j), d, sd)
        pltpu.async_copy(h_hbm.at[_sidx(j)], r, sg)

    def _finish_and_scatter(j, o):
        r, d, sg, ss, sd = slots[o]
        pltpu.make_async_copy(h_hbm.at[_sidx(j)], r, sg).wait()
        pltpu.make_async_copy(_dslice(j), d, sd).wait()
        pltpu.async_copy(r, acc_sh.at[d], ss, add=True)

    def _drain_scatter(o):
        r, d, sg, ss, sd = slots[o]
        pltpu.make_async_copy(r, acc_sh.at[d], ss).wait()

    pltpu.sync_copy(
        src_hbm.at[pl.ds(wid * CHUNKS_PER_TILE * EDGE_CHUNK,
                         CHUNKS_PER_TILE * EDGE_CHUNK)], src_v)
    _start_load(0, 0)
    _start_load(1, 1)

    _zero_vmem_2d(r2, ROW_CHUNK, D)

    @pl.loop(sid, N_ROW_CHUNKS, step=NS)
    def _(j):
        pltpu.sync_copy(r2.at[pl.ds(0, ROW_CHUNK)],
                        acc_sh.at[pl.ds(j * ROW_CHUNK, ROW_CHUNK)])

    plsc.subcore_barrier()

    _finish_and_scatter(0, 0)
    _start_load(2, 2)
    _finish_and_scatter(1, 1)
    _start_load(3, 3)
    _finish_and_scatter(2, 2)
    _drain_scatter(0)
    _start_load(4, 0)
    _finish_and_scatter(3, 3)
    _drain_scatter(1)
    _start_load(5, 1)

    @pl.loop(1, CHUNKS_PER_TILE // 4 - 1)
    def _(t):
        j0 = 4 * t
        for o in range(4):
            j = j0 + o
            _finish_and_scatter(j, o)
            _drain_scatter((o + 2) % 4)
            _start_load(j + 2, (o + 2) % 4)

    jl = CHUNKS_PER_TILE - 4
    _finish_and_scatter(jl, 0)
    _drain_scatter(2)
    _start_load(jl + 2, 2)
    _finish_and_scatter(jl + 1, 1)
    _drain_scatter(3)
    _start_load(jl + 3, 3)
    _finish_and_scatter(jl + 2, 2)
    _drain_scatter(0)
    _finish_and_scatter(jl + 3, 3)
    _drain_scatter(1)
    _drain_scatter(2)
    _drain_scatter(3)

    plsc.subcore_barrier()

    @pl.loop(sid, N_OUT_CHUNKS, step=NS)
    def _(j):
        pltpu.sync_copy(acc_sh.at[pl.ds(j * OUT_CHUNK, OUT_CHUNK)],
                        out_hbm.at[cid, pl.ds(j * OUT_CHUNK, OUT_CHUNK)])


def _tc_layer_body(h_ref, p_ref, wrel_ref, wroot_ref, brel_ref,
                   gamma_ref, beta_ref, o_ref):
    agg = p_ref[0] + p_ref[1]
    out = (jnp.dot(agg, wrel_ref[...], preferred_element_type=jnp.float32)
           + jnp.dot(h_ref[...], wroot_ref[...],
                     preferred_element_type=jnp.float32)
           + brel_ref[...])
    mu = jnp.mean(out, axis=0, keepdims=True)
    var = jnp.mean((out - mu) ** 2, axis=0, keepdims=True)
    normed = (out - mu) * lax.rsqrt(var + 1e-5) * gamma_ref[...] + beta_ref[...]
    o_ref[...] = jnp.maximum(normed, 0.0)


_tc_layer = pl.pallas_call(
    _tc_layer_body,
    out_shape=jax.ShapeDtypeStruct((N, D), jnp.float32),
)


@functools.partial(
    pl.kernel,
    out_type=[jax.ShapeDtypeStruct((NC, S, D), jnp.float32),
              jax.ShapeDtypeStruct((NC, S, D), jnp.float32)],
    mesh=_mesh,
    scratch_types=[
        pltpu.VMEM((G,), jnp.int32),
        pltpu.VMEM((G,), jnp.int32),
        pltpu.VMEM((POOL_CHUNK,), jnp.int32),
        pltpu.VMEM((POOL_CHUNK,), jnp.int32),
        pltpu.VMEM((POOL_CHUNK,), jnp.int32),
        pltpu.VMEM((POOL_CHUNK, D), jnp.float32),
        pltpu.VMEM((POOL_CHUNK, D), jnp.float32),
        pltpu.VMEM_SHARED((S, D), jnp.float32),
        pltpu.VMEM_SHARED((S, D), jnp.float32),
    ],
    compiler_params=_sc_params,
)
def _sc_pool(h_hbm, batch_hbm, sb_hbm, ns_hbm, sum_hbm, cnt_hbm,
             ns_v, offs_v, bt_v, sb_v, id_v, rows_v, ones_v,
             acc_sh, cnt_sh):
    cid = lax.axis_index("c")
    sid = lax.axis_index("s")
    wid = sid * NC + cid

    pltpu.sync_copy(ns_hbm, ns_v)
    carry = jnp.int32(0)
    for k in range(G // 16):
        v = ns_v[pl.ds(k * 16, 16)]
        incl = plsc.cumsum(v)
        offs_v[pl.ds(k * 16, 16)] = incl - v + carry
        carry = carry + jnp.sum(v)

    _zero_vmem_2d(rows_v, POOL_CHUNK, D)
    one = jnp.ones((16,), jnp.float32)

    @pl.loop(0, POOL_CHUNK)
    def _(r):
        @pl.loop(0, D // 16)
        def _(c):
            ones_v[r, pl.ds(c * 16, 16)] = one

    pbase = sid * POOL_ROWS_PER_SUBCORE
    pltpu.sync_copy(rows_v.at[pl.ds(0, POOL_ROWS_PER_SUBCORE)],
                    acc_sh.at[pl.ds(pbase, POOL_ROWS_PER_SUBCORE)])
    pltpu.sync_copy(rows_v.at[pl.ds(0, POOL_ROWS_PER_SUBCORE)],
                    cnt_sh.at[pl.ds(pbase, POOL_ROWS_PER_SUBCORE)])
    plsc.subcore_barrier()

    @pl.loop(wid, N_POOL_CHUNKS, step=NW)
    def _(i):
        nbase = i * POOL_CHUNK
        pltpu.sync_copy(batch_hbm.at[pl.ds(nbase, POOL_CHUNK)], bt_v)
        pltpu.sync_copy(sb_hbm.at[pl.ds(nbase, POOL_CHUNK)], sb_v)
        for k in range(POOL_CHUNK // 16):
            idx16 = bt_v[pl.ds(k * 16, 16)]
            off16 = plsc.load_gather(offs_v, [idx16])
            id_v[pl.ds(k * 16, 16)] = sb_v[pl.ds(k * 16, 16)] + off16
        pltpu.sync_copy(h_hbm.at[pl.ds(nbase, POOL_CHUNK)], rows_v)
        pltpu.sync_copy(rows_v, acc_sh.at[id_v], add=True)
        pltpu.sync_copy(ones_v, cnt_sh.at[id_v], add=True)

    plsc.subcore_barrier()
    pltpu.sync_copy(acc_sh.at[pl.ds(pbase, POOL_ROWS_PER_SUBCORE)],
                    sum_hbm.at[cid, pl.ds(pbase, POOL_ROWS_PER_SUBCORE)])
    pltpu.sync_copy(cnt_sh.at[pl.ds(pbase, POOL_ROWS_PER_SUBCORE)],
                    cnt_hbm.at[cid, pl.ds(pbase, POOL_ROWS_PER_SUBCORE)])


def _tc_finalize_body(s_ref, c_ref, o_ref):
    s = s_ref[0] + s_ref[1]
    c = c_ref[0] + c_ref[1]
    o_ref[...] = s / jnp.maximum(c[:, 0:1], 1.0)


_tc_finalize = pl.pallas_call(
    _tc_finalize_body,
    out_shape=jax.ShapeDtypeStruct((S, D), jnp.float32),
)


def kernel(x, edge_index, edge_attr, batch, num_subgraphs, subgraph_batch,
           Wroot, Wrel, brel, gamma, beta):
    pad = E_PAD - E
    src = jnp.concatenate([edge_index[0], jnp.zeros((pad,), jnp.int32)])
    dst = jnp.concatenate([edge_index[1], jnp.full((pad,), N, jnp.int32)])
    h = x
    for l in range(L):
        partials = _sc_segment_sum(h, src, dst)
        h = _tc_layer(h, partials, Wrel[l], Wroot[l],
                      brel[l].reshape(1, D), gamma[l].reshape(1, D),
                      beta[l].reshape(1, D))
    sums, cnts = _sc_pool(h, batch, subgraph_batch, num_subgraphs)
    return _tc_finalize(sums, cnts)


# speedup vs baseline: 1.0843x; 1.0788x over previous
import dataclasses
import functools

import jax
import jax.numpy as jnp
from jax import lax
from jax.experimental import pallas as pl
from jax.experimental.pallas import tpu as pltpu
from jax.experimental.pallas import tpu_sc as plsc

N = 10000
E = 320000
D = 128
L = 5
G = 64
S = 512

NC = 2
NS = 16
NW = NC * NS

EDGE_CHUNK = 96
CHUNKS_PER_TILE = 105
E_PAD = NW * CHUNKS_PER_TILE * EDGE_CHUNK
ROW_CHUNK = 80
N_ROW_CHUNKS = N // ROW_CHUNK
OUT_CHUNK = 200
N_OUT_CHUNKS = N // OUT_CHUNK
N_ACC = N + 8

POOL_CHUNK = 80
N_POOL_CHUNKS = N // POOL_CHUNK
POOL_ROWS_PER_SUBCORE = S // NS

_mesh = plsc.VectorSubcoreMesh(core_axis_name="c", subcore_axis_name="s")

_sc_params = pltpu.CompilerParams()
if "needs_layout_passes" in pltpu.CompilerParams.__dataclass_fields__:
    _sc_params = dataclasses.replace(_sc_params, needs_layout_passes=False)


def _zero_vmem_2d(ref, rows, cols):
    z = jnp.zeros((16,), jnp.float32)

    @pl.loop(0, rows)
    def _(r):
        @pl.loop(0, cols // 16)
        def _(c):
            ref[r, pl.ds(c * 16, 16)] = z


@functools.partial(
    pl.kernel,
    out_type=jax.ShapeDtypeStruct((NC, N, D), jnp.float32),
    mesh=_mesh,
    scratch_types=[
        pltpu.VMEM((CHUNKS_PER_TILE * EDGE_CHUNK,), jnp.int32),
        pltpu.VMEM((EDGE_CHUNK,), jnp.int32),
        pltpu.VMEM((EDGE_CHUNK,), jnp.int32),
        pltpu.VMEM((EDGE_CHUNK,), jnp.int32),
        pltpu.VMEM((EDGE_CHUNK, D), jnp.float32),
        pltpu.VMEM((EDGE_CHUNK, D), jnp.float32),
        pltpu.VMEM((EDGE_CHUNK, D), jnp.float32),
        pltpu.VMEM_SHARED((N_ACC, D), jnp.float32),
    ] + [pltpu.SemaphoreType.DMA] * 10,
)
def _sc_segment_sum(h_hbm, src_hbm, dst_hbm, out_hbm,
                    src_v, d0, d1, d2, r0, r1, r2, acc_sh,
                    sg0, sg1, sg2, ss0, ss1, ss2, sd0, sd1, sd2, sz):
    cid = lax.axis_index("c")
    sid = lax.axis_index("s")
    wid = sid * NC + cid
    tbase = wid * CHUNKS_PER_TILE

    def _sidx(j):
        return src_v.at[pl.ds(j * EDGE_CHUNK, EDGE_CHUNK)]

    def _dslice(j):
        return dst_hbm.at[pl.ds((tbase + j) * EDGE_CHUNK, EDGE_CHUNK)]

    slots = ((r0, d0, sg0, ss0, sd0),
             (r1, d1, sg1, ss1, sd1),
             (r2, d2, sg2, ss2, sd2))

    def _start_load(j, o):
        r, d, sg, ss, sd = slots[o]
        pltpu.async_copy(_dslice(j), d, sd)
        pltpu.async_copy(h_hbm.at[_sidx(j)], r, sg)

    def _finish_and_scatter(j, o):
        r, d, sg, ss, sd = slots[o]
        pltpu.make_async_copy(h_hbm.at[_sidx(j)], r, sg).wait()
        pltpu.make_async_copy(_dslice(j), d, sd).wait()
        pltpu.async_copy(r, acc_sh.at[d], ss, add=True)

    def _drain_scatter(o):
        r, d, sg, ss, sd = slots[o]
        pltpu.make_async_copy(r, acc_sh.at[d], ss).wait()

    pltpu.sync_copy(
        src_hbm.at[pl.ds(wid * CHUNKS_PER_TILE * EDGE_CHUNK,
                         CHUNKS_PER_TILE * EDGE_CHUNK)], src_v)
    _start_load(0, 0)
    _start_load(1, 1)

    _zero_vmem_2d(r2, ROW_CHUNK, D)
    zsrc = r2.at[pl.ds(0, ROW_CHUNK)]

    @pl.loop(sid, N_ROW_CHUNKS, step=NS)
    def _(j):
        pltpu.async_copy(zsrc, acc_sh.at[pl.ds(j * ROW_CHUNK, ROW_CHUNK)], sz)

    @pl.loop(sid, N_ROW_CHUNKS, step=NS)
    def _(j):
        pltpu.make_async_copy(
            zsrc, acc_sh.at[pl.ds(j * ROW_CHUNK, ROW_CHUNK)], sz).wait()

    plsc.subcore_barrier()

    _start_load(2, 2)

    @pl.loop(0, CHUNKS_PER_TILE // 3 - 1)
    def _(t):
        j0 = 3 * t
        for o in range(3):
            _finish_and_scatter(j0 + o, o)
            _drain_scatter(o)
            _start_load(j0 + 3 + o, o)

    jlast = CHUNKS_PER_TILE - 3
    for o in range(3):
        _finish_and_scatter(jlast + o, o)
        _drain_scatter(o)

    plsc.subcore_barrier()

    @pl.loop(sid, N_OUT_CHUNKS, step=NS)
    def _(j):
        pltpu.async_copy(acc_sh.at[pl.ds(j * OUT_CHUNK, OUT_CHUNK)],
                         out_hbm.at[cid, pl.ds(j * OUT_CHUNK, OUT_CHUNK)], sz)

    @pl.loop(sid, N_OUT_CHUNKS, step=NS)
    def _(j):
        pltpu.make_async_copy(
            acc_sh.at[pl.ds(j * OUT_CHUNK, OUT_CHUNK)],
            out_hbm.at[cid, pl.ds(j * OUT_CHUNK, OUT_CHUNK)], sz).wait()


def _tc_layer_body(h_ref, p_ref, wrel_ref, wroot_ref, brel_ref,
                   gamma_ref, beta_ref, o_ref):
    agg = p_ref[0] + p_ref[1]
    out = (jnp.dot(agg, wrel_ref[...], preferred_element_type=jnp.float32)
           + jnp.dot(h_ref[...], wroot_ref[...],
                     preferred_element_type=jnp.float32)
           + brel_ref[...])
    mu = jnp.mean(out, axis=0, keepdims=True)
    var = jnp.mean((out - mu) ** 2, axis=0, keepdims=True)
    normed = (out - mu) * lax.rsqrt(var + 1e-5) * gamma_ref[...] + beta_ref[...]
    o_ref[...] = jnp.maximum(normed, 0.0)


_tc_layer = pl.pallas_call(
    _tc_layer_body,
    out_shape=jax.ShapeDtypeStruct((N, D), jnp.float32),
)


@functools.partial(
    pl.kernel,
    out_type=[jax.ShapeDtypeStruct((NC, S, D), jnp.float32),
              jax.ShapeDtypeStruct((NC, S, D), jnp.float32)],
    mesh=_mesh,
    scratch_types=[
        pltpu.VMEM((G,), jnp.int32),
        pltpu.VMEM((G,), jnp.int32),
        pltpu.VMEM((POOL_CHUNK,), jnp.int32),
        pltpu.VMEM((POOL_CHUNK,), jnp.int32),
        pltpu.VMEM((POOL_CHUNK,), jnp.int32),
        pltpu.VMEM((POOL_CHUNK, D), jnp.float32),
        pltpu.VMEM((POOL_CHUNK, D), jnp.float32),
        pltpu.VMEM_SHARED((S, D), jnp.float32),
        pltpu.VMEM_SHARED((S, D), jnp.float32),
    ],
    compiler_params=_sc_params,
)
def _sc_pool(h_hbm, batch_hbm, sb_hbm, ns_hbm, sum_hbm, cnt_hbm,
             ns_v, offs_v, bt_v, sb_v, id_v, rows_v, ones_v,
             acc_sh, cnt_sh):
    cid = lax.axis_index("c")
    sid = lax.axis_index("s")
    wid = sid * NC + cid

    pltpu.sync_copy(ns_hbm, ns_v)
    carry = jnp.int32(0)
    for k in range(G // 16):
        v = ns_v[pl.ds(k * 16, 16)]
        incl = plsc.cumsum(v)
        offs_v[pl.ds(k * 16, 16)] = incl - v + carry
        carry = carry + jnp.sum(v)

    _zero_vmem_2d(rows_v, POOL_CHUNK, D)
    one = jnp.ones((16,), jnp.float32)

    @pl.loop(0, POOL_CHUNK)
    def _(r):
        @pl.loop(0, D // 16)
        def _(c):
            ones_v[r, pl.ds(c * 16, 16)] = one

    pbase = sid * POOL_ROWS_PER_SUBCORE
    pltpu.sync_copy(rows_v.at[pl.ds(0, POOL_ROWS_PER_SUBCORE)],
                    acc_sh.at[pl.ds(pbase, POOL_ROWS_PER_SUBCORE)])
    pltpu.sync_copy(rows_v.at[pl.ds(0, POOL_ROWS_PER_SUBCORE)],
                    cnt_sh.at[pl.ds(pbase, POOL_ROWS_PER_SUBCORE)])
    plsc.subcore_barrier()

    @pl.loop(wid, N_POOL_CHUNKS, step=NW)
    def _(i):
        nbase = i * POOL_CHUNK
        pltpu.sync_copy(batch_hbm.at[pl.ds(nbase, POOL_CHUNK)], bt_v)
        pltpu.sync_copy(sb_hbm.at[pl.ds(nbase, POOL_CHUNK)], sb_v)
        for k in range(POOL_CHUNK // 16):
            idx16 = bt_v[pl.ds(k * 16, 16)]
            off16 = plsc.load_gather(offs_v, [idx16])
            id_v[pl.ds(k * 16, 16)] = sb_v[pl.ds(k * 16, 16)] + off16
        pltpu.sync_copy(h_hbm.at[pl.ds(nbase, POOL_CHUNK)], rows_v)
        pltpu.sync_copy(rows_v, acc_sh.at[id_v], add=True)
        pltpu.sync_copy(ones_v, cnt_sh.at[id_v], add=True)

    plsc.subcore_barrier()
    pltpu.sync_copy(acc_sh.at[pl.ds(pbase, POOL_ROWS_PER_SUBCORE)],
                    sum_hbm.at[cid, pl.ds(pbase, POOL_ROWS_PER_SUBCORE)])
    pltpu.sync_copy(cnt_sh.at[pl.ds(pbase, POOL_ROWS_PER_SUBCORE)],
                    cnt_hbm.at[cid, pl.ds(pbase, POOL_ROWS_PER_SUBCORE)])


def _tc_finalize_body(s_ref, c_ref, o_ref):
    s = s_ref[0] + s_ref[1]
    c = c_ref[0] + c_ref[1]
    o_ref[...] = s / jnp.maximum(c[:, 0:1], 1.0)


_tc_finalize = pl.pallas_call(
    _tc_finalize_body,
    out_shape=jax.ShapeDtypeStruct((S, D), jnp.float32),
)


def kernel(x, edge_index, edge_attr, batch, num_subgraphs, subgraph_batch,
           Wroot, Wrel, brel, gamma, beta):
    pad = E_PAD - E
    src = jnp.concatenate([edge_index[0], jnp.zeros((pad,), jnp.int32)])
    dst = jnp.concatenate([edge_index[1], jnp.full((pad,), N, jnp.int32)])
    h = x
    for l in range(L):
        partials = _sc_segment_sum(h, src, dst)
        h = _tc_layer(h, partials, Wrel[l], Wroot[l],
                      brel[l].reshape(1, D), gamma[l].reshape(1, D),
                      beta[l].reshape(1, D))
    sums, cnts = _sc_pool(h, batch, subgraph_batch, num_subgraphs)
    return _tc_finalize(sums, cnts)


# speedup vs baseline: 2.1256x; 1.9604x over previous
import dataclasses
import functools

import jax
import jax.numpy as jnp
from jax import lax
from jax.experimental import pallas as pl
from jax.experimental.pallas import tpu as pltpu
from jax.experimental.pallas import tpu_sc as plsc

N = 10000
E = 320000
D = 128
L = 5
G = 64
S = 512

NC = 2
NS = 16
NW = NC * NS

EDGE_CHUNK = 80
CHUNKS_PER_TILE = 125
ROW_CHUNK = 80
N_ROW_CHUNKS = N // ROW_CHUNK
OUT_CHUNK = 200
N_OUT_CHUNKS = N // OUT_CHUNK
N_ACC = N

POOL_CHUNK = 80
N_POOL_CHUNKS = N // POOL_CHUNK
POOL_ROWS_PER_SUBCORE = S // NS

_mesh = plsc.VectorSubcoreMesh(core_axis_name="c", subcore_axis_name="s")

_sc_params = pltpu.CompilerParams()
if "needs_layout_passes" in pltpu.CompilerParams.__dataclass_fields__:
    _sc_params = dataclasses.replace(_sc_params, needs_layout_passes=False)


@functools.partial(
    pl.kernel,
    out_type=jax.ShapeDtypeStruct((NC, N, D), jnp.float32),
    mesh=_mesh,
    scratch_types=[
        pltpu.VMEM((CHUNKS_PER_TILE * EDGE_CHUNK,), jnp.int32),
        pltpu.VMEM((EDGE_CHUNK,), jnp.int32),
        pltpu.VMEM((EDGE_CHUNK,), jnp.int32),
        pltpu.VMEM((EDGE_CHUNK,), jnp.int32),
        pltpu.VMEM((EDGE_CHUNK, D), jnp.float32),
        pltpu.VMEM((EDGE_CHUNK, D), jnp.float32),
        pltpu.VMEM((EDGE_CHUNK, D), jnp.float32),
        pltpu.VMEM_SHARED((N_ACC, D), jnp.float32),
    ] + [pltpu.SemaphoreType.DMA] * 10,
)
def _sc_segment_sum(h_hbm, src_hbm, dst_hbm, zeros_hbm, out_hbm,
                    src_v, d0, d1, d2, r0, r1, r2, acc_sh,
                    sg0, sg1, sg2, ss0, ss1, ss2, sd0, sd1, sd2, sz):
    cid = lax.axis_index("c")
    sid = lax.axis_index("s")
    wid = sid * NC + cid
    tbase = wid * CHUNKS_PER_TILE

    def _sidx(j):
        return src_v.at[pl.ds(j * EDGE_CHUNK, EDGE_CHUNK)]

    def _dslice(j):
        return dst_hbm.at[pl.ds((tbase + j) * EDGE_CHUNK, EDGE_CHUNK)]

    slots = ((r0, d0, sg0, ss0, sd0),
             (r1, d1, sg1, ss1, sd1),
             (r2, d2, sg2, ss2, sd2))

    def _start_load(j, o):
        r, d, sg, ss, sd = slots[o]
        pltpu.async_copy(_dslice(j), d, sd)
        pltpu.async_copy(h_hbm.at[_sidx(j)], r, sg)

    def _finish_and_scatter(j, o):
        r, d, sg, ss, sd = slots[o]
        pltpu.make_async_copy(h_hbm.at[_sidx(j)], r, sg).wait()
        pltpu.make_async_copy(_dslice(j), d, sd).wait()
        pltpu.async_copy(r, acc_sh.at[d], ss, add=True)

    def _drain_scatter(o):
        r, d, sg, ss, sd = slots[o]
        pltpu.make_async_copy(r, acc_sh.at[d], ss).wait()

    pltpu.sync_copy(
        src_hbm.at[pl.ds(wid * CHUNKS_PER_TILE * EDGE_CHUNK,
                         CHUNKS_PER_TILE * EDGE_CHUNK)], src_v)
    _start_load(0, 0)
    _start_load(1, 1)

    pltpu.sync_copy(zeros_hbm, r2)
    zsrc = r2.at[pl.ds(0, ROW_CHUNK)]

    @pl.loop(sid, N_ROW_CHUNKS, step=NS)
    def _(j):
        pltpu.async_copy(zsrc, acc_sh.at[pl.ds(j * ROW_CHUNK, ROW_CHUNK)], sz)

    @pl.loop(sid, N_ROW_CHUNKS, step=NS)
    def _(j):
        pltpu.make_async_copy(
            zsrc, acc_sh.at[pl.ds(j * ROW_CHUNK, ROW_CHUNK)], sz).wait()

    plsc.subcore_barrier()

    _start_load(2, 2)

    NT3 = (CHUNKS_PER_TILE - 5) // 3

    @pl.loop(0, NT3)
    def _(t):
        j0 = 3 * t
        for o in range(3):
            _finish_and_scatter(j0 + o, o)
            _drain_scatter(o)
            _start_load(j0 + 3 + o, o)

    je = 3 * NT3
    _finish_and_scatter(je, 0)
    _drain_scatter(0)
    _start_load(je + 3, 0)
    _finish_and_scatter(je + 1, 1)
    _drain_scatter(1)
    _start_load(je + 4, 1)
    _finish_and_scatter(je + 2, 2)
    _drain_scatter(2)
    _finish_and_scatter(je + 3, 0)
    _drain_scatter(0)
    _finish_and_scatter(je + 4, 1)
    _drain_scatter(1)

    plsc.subcore_barrier()

    @pl.loop(sid, N_OUT_CHUNKS, step=NS)
    def _(j):
        pltpu.async_copy(acc_sh.at[pl.ds(j * OUT_CHUNK, OUT_CHUNK)],
                         out_hbm.at[cid, pl.ds(j * OUT_CHUNK, OUT_CHUNK)], sz)

    @pl.loop(sid, N_OUT_CHUNKS, step=NS)
    def _(j):
        pltpu.make_async_copy(
            acc_sh.at[pl.ds(j * OUT_CHUNK, OUT_CHUNK)],
            out_hbm.at[cid, pl.ds(j * OUT_CHUNK, OUT_CHUNK)], sz).wait()


def _tc_layer_body(h_ref, p_ref, wrel_ref, wroot_ref, brel_ref,
                   gamma_ref, beta_ref, o_ref):
    agg = p_ref[0] + p_ref[1]
    out = (jnp.dot(agg, wrel_ref[...], preferred_element_type=jnp.float32)
           + jnp.dot(h_ref[...], wroot_ref[...],
                     preferred_element_type=jnp.float32)
           + brel_ref[...])
    mu = jnp.mean(out, axis=0, keepdims=True)
    var = jnp.mean((out - mu) ** 2, axis=0, keepdims=True)
    normed = (out - mu) * lax.rsqrt(var + 1e-5) * gamma_ref[...] + beta_ref[...]
    o_ref[...] = jnp.maximum(normed, 0.0)


_tc_layer = pl.pallas_call(
    _tc_layer_body,
    out_shape=jax.ShapeDtypeStruct((N, D), jnp.float32),
)


@functools.partial(
    pl.kernel,
    out_type=[jax.ShapeDtypeStruct((NC, S, D), jnp.float32),
              jax.ShapeDtypeStruct((NC, S, D), jnp.float32)],
    mesh=_mesh,
    scratch_types=[
        pltpu.VMEM((G,), jnp.int32),
        pltpu.VMEM((G,), jnp.int32),
        pltpu.VMEM((POOL_CHUNK,), jnp.int32),
        pltpu.VMEM((POOL_CHUNK,), jnp.int32),
        pltpu.VMEM((POOL_CHUNK,), jnp.int32),
        pltpu.VMEM((POOL_CHUNK, D), jnp.float32),
        pltpu.VMEM((POOL_CHUNK, D), jnp.float32),
        pltpu.VMEM_SHARED((S, D), jnp.float32),
        pltpu.VMEM_SHARED((S, D), jnp.float32),
    ],
    compiler_params=_sc_params,
)
def _sc_pool(h_hbm, batch_hbm, sb_hbm, ns_hbm, zeros_hbm, ones_hbm,
             sum_hbm, cnt_hbm,
             ns_v, offs_v, bt_v, sb_v, id_v, rows_v, ones_v,
             acc_sh, cnt_sh):
    cid = lax.axis_index("c")
    sid = lax.axis_index("s")
    wid = sid * NC + cid

    pltpu.sync_copy(ns_hbm, ns_v)
    carry = jnp.int32(0)
    for k in range(G // 16):
        v = ns_v[pl.ds(k * 16, 16)]
        incl = plsc.cumsum(v)
        offs_v[pl.ds(k * 16, 16)] = incl - v + carry
        carry = carry + jnp.sum(v)

    pltpu.sync_copy(zeros_hbm, rows_v)
    pltpu.sync_copy(ones_hbm, ones_v)

    pbase = sid * POOL_ROWS_PER_SUBCORE
    pltpu.sync_copy(rows_v.at[pl.ds(0, POOL_ROWS_PER_SUBCORE)],
                    acc_sh.at[pl.ds(pbase, POOL_ROWS_PER_SUBCORE)])
    pltpu.sync_copy(rows_v.at[pl.ds(0, POOL_ROWS_PER_SUBCORE)],
                    cnt_sh.at[pl.ds(pbase, POOL_ROWS_PER_SUBCORE)])
    plsc.subcore_barrier()

    @pl.loop(wid, N_POOL_CHUNKS, step=NW)
    def _(i):
        nbase = i * POOL_CHUNK
        pltpu.sync_copy(batch_hbm.at[pl.ds(nbase, POOL_CHUNK)], bt_v)
        pltpu.sync_copy(sb_hbm.at[pl.ds(nbase, POOL_CHUNK)], sb_v)
        for k in range(POOL_CHUNK // 16):
            idx16 = bt_v[pl.ds(k * 16, 16)]
            off16 = plsc.load_gather(offs_v, [idx16])
            id_v[pl.ds(k * 16, 16)] = sb_v[pl.ds(k * 16, 16)] + off16
        pltpu.sync_copy(h_hbm.at[pl.ds(nbase, POOL_CHUNK)], rows_v)
        pltpu.sync_copy(rows_v, acc_sh.at[id_v], add=True)
        pltpu.sync_copy(ones_v, cnt_sh.at[id_v], add=True)

    plsc.subcore_barrier()
    pltpu.sync_copy(acc_sh.at[pl.ds(pbase, POOL_ROWS_PER_SUBCORE)],
                    sum_hbm.at[cid, pl.ds(pbase, POOL_ROWS_PER_SUBCORE)])
    pltpu.sync_copy(cnt_sh.at[pl.ds(pbase, POOL_ROWS_PER_SUBCORE)],
                    cnt_hbm.at[cid, pl.ds(pbase, POOL_ROWS_PER_SUBCORE)])


def _tc_finalize_body(s_ref, c_ref, o_ref):
    s = s_ref[0] + s_ref[1]
    c = c_ref[0] + c_ref[1]
    o_ref[...] = s / jnp.maximum(c[:, 0:1], 1.0)


_tc_finalize = pl.pallas_call(
    _tc_finalize_body,
    out_shape=jax.ShapeDtypeStruct((S, D), jnp.float32),
)


def kernel(x, edge_index, edge_attr, batch, num_subgraphs, subgraph_batch,
           Wroot, Wrel, brel, gamma, beta):
    src = edge_index[0]
    dst = edge_index[1]
    zeros2d = jnp.zeros((ROW_CHUNK, D), jnp.float32)
    ones2d = jnp.ones((POOL_CHUNK, D), jnp.float32)
    h = x
    for l in range(L):
        partials = _sc_segment_sum(h, src, dst, zeros2d)
        h = _tc_layer(h, partials, Wrel[l], Wroot[l],
                      brel[l].reshape(1, D), gamma[l].reshape(1, D),
                      beta[l].reshape(1, D))
    sums, cnts = _sc_pool(h, batch, subgraph_batch, num_subgraphs,
                          zeros2d, ones2d)
    return _tc_finalize(sums, cnts)
